# baseline (speedup 1.0000x reference)
import functools

import jax
import jax.numpy as jnp
from jax import lax
from jax.experimental import pallas as pl
from jax.experimental.pallas import tpu as pltpu
from jax.experimental.pallas import tpu_sc as plsc

NUM_CORES = 2
NUM_SUBCORES = 16
NW = NUM_CORES * NUM_SUBCORES
LANES = 16
B = 16384
D = 32
BPW = B // NW
IDX_CHUNK = 128
NCHUNK = BPW // IDX_CHUNK
NGROUP = BPW // LANES


def _mf_body(uidx_hbm, iidx_hbm, utab_hbm, itab_hbm, wb_hbm, out_hbm,
             uidx_v, iidx_v, urows_v, irows_v, wb_v, out_v, usem, isem):
    wid = lax.axis_index("s") * NUM_CORES + lax.axis_index("c")
    base = wid * BPW

    pltpu.sync_copy(uidx_hbm.at[wid], uidx_v)
    pltpu.sync_copy(iidx_hbm.at[wid], iidx_v)
    pltpu.sync_copy(wb_hbm, wb_v)

    copies = []
    for j in range(NCHUNK):
        dst = pl.ds(j * IDX_CHUNK, IDX_CHUNK)
        copies.append(pltpu.async_copy(utab_hbm.at[uidx_v.at[j]],
                                       urows_v.at[dst], usem))
        copies.append(pltpu.async_copy(itab_hbm.at[iidx_v.at[j]],
                                       irows_v.at[dst], isem))
    for c in copies:
        c.wait()

    w = wb_v[0, :]
    b = wb_v[1, :]

    def group(g, carry):
        row0 = pl.multiple_of(g * LANES, LANES)
        acc = jnp.zeros((16,), jnp.float32)
        for v in range(LANES):
            sl = pl.ds(row0 + v, 1)
            acc = acc + urows_v[row0 + v, :] * irows_v[row0 + v, :]
        s = acc * w + b
        y = 1.0 / (1.0 + jnp.exp(-s))
        out_v[pl.ds(row0, LANES)] = y
        return carry

    lax.fori_loop(0, NGROUP, group, 0)
    pltpu.sync_copy(out_v, out_hbm.at[pl.ds(base, BPW)])


@jax.jit
def _mf_call(uidx, iidx, user_table, item_table, wb):
    mesh = plsc.VectorSubcoreMesh(core_axis_name="c", subcore_axis_name="s",
                                  num_cores=NUM_CORES,
                                  num_subcores=NUM_SUBCORES)
    fn = pl.kernel(
        _mf_body,
        out_type=jax.ShapeDtypeStruct((B,), jnp.float32),
        mesh=mesh,
        compiler_params=pltpu.CompilerParams(needs_layout_passes=False,
                                             use_tc_tiling_on_sc=False),
        scratch_types=[
            pltpu.VMEM((NCHUNK, IDX_CHUNK), jnp.int32),
            pltpu.VMEM((NCHUNK, IDX_CHUNK), jnp.int32),
            pltpu.VMEM((BPW, LANES), jnp.float32),
            pltpu.VMEM((BPW, LANES), jnp.float32),
            pltpu.VMEM((2, 16), jnp.float32),
            pltpu.VMEM((BPW,), jnp.float32),
            pltpu.SemaphoreType.DMA,
            pltpu.SemaphoreType.DMA,
        ],
    )
    return fn(uidx, iidx, user_table, item_table, wb)


def kernel(user_idx, item_idx, user_table, item_table, W_aff, b_aff):
    uidx = (user_idx.reshape(NW, NCHUNK, IDX_CHUNK).astype(jnp.int32)
            & jnp.int32(0x000FFFFF))
    iidx = (item_idx.reshape(NW, NCHUNK, IDX_CHUNK).astype(jnp.int32)
            & jnp.int32(0x000FFFFF))
    wb = jnp.stack([jnp.full((16,), W_aff[0, 0], jnp.float32),
                    jnp.full((16,), b_aff[0], jnp.float32)])
    ut = user_table.T.reshape(2 * 1000000, 16)
    it = item_table.T.reshape(2 * 1000000, 16)
    return _mf_call(uidx, iidx, ut, it, wb)


# speedup vs baseline: 5.6151x; 5.6151x over previous
import functools

import jax
import jax.numpy as jnp
from jax import lax
from jax.experimental import pallas as pl
from jax.experimental.pallas import tpu as pltpu
from jax.experimental.pallas import tpu_sc as plsc

NUM_CORES = 2
NUM_SUBCORES = 16
NW = NUM_CORES * NUM_SUBCORES
LANES = 16
B = 16384
D = 32
BPW = B // NW
IDX_CHUNK = 128
NCHUNK = BPW // IDX_CHUNK
NGROUP = BPW // LANES


def _mf_body(uidx_hbm, iidx_hbm, utab_hbm, itab_hbm, wb_hbm, out_hbm,
             uidx_v, iidx_v, urows_v, irows_v, wb_v, out_v, usem, isem):
    wid = lax.axis_index("s") * NUM_CORES + lax.axis_index("c")
    base = wid * BPW

    pltpu.sync_copy(uidx_hbm.at[wid], uidx_v)
    pltpu.sync_copy(iidx_hbm.at[wid], iidx_v)
    pltpu.sync_copy(wb_hbm, wb_v)

    copies = []
    for j in range(NCHUNK):
        dst = pl.ds(j * IDX_CHUNK, IDX_CHUNK)
        copies.append(pltpu.async_copy(utab_hbm.at[uidx_v.at[j]],
                                       urows_v.at[dst], usem))
        copies.append(pltpu.async_copy(itab_hbm.at[iidx_v.at[j]],
                                       irows_v.at[dst], isem))
    for c in copies:
        c.wait()

    w = wb_v[0, :]
    b = wb_v[1, :]
    lane = lax.iota(jnp.int32, 16)

    def group(g, carry):
        row0 = pl.multiple_of(g * LANES, LANES)
        rows = row0 + lane
        acc = jnp.zeros((16,), jnp.float32)
        for d in range(D):
            cols = jnp.full((16,), d, jnp.int32)
            uv = plsc.load_gather(urows_v, [rows, cols])
            iv = plsc.load_gather(irows_v, [rows, cols])
            acc = acc + uv * iv
        s = acc * w + b
        y = 1.0 / (1.0 + jnp.exp(-s))
        out_v[pl.ds(row0, LANES)] = y
        return carry

    lax.fori_loop(0, NGROUP, group, 0)
    pltpu.sync_copy(out_v, out_hbm.at[pl.ds(base, BPW)])


@functools.partial(jax.jit, static_argnames=())
def _mf_call(uidx, iidx, user_table, item_table, wb):
    mesh = plsc.VectorSubcoreMesh(core_axis_name="c", subcore_axis_name="s",
                                  num_cores=NUM_CORES,
                                  num_subcores=NUM_SUBCORES)
    fn = pl.kernel(
        _mf_body,
        out_type=jax.ShapeDtypeStruct((B,), jnp.float32),
        mesh=mesh,
        compiler_params=pltpu.CompilerParams(needs_layout_passes=False,
                                             use_tc_tiling_on_sc=False),
        scratch_types=[
            pltpu.VMEM((NCHUNK, IDX_CHUNK), jnp.int32),
            pltpu.VMEM((NCHUNK, IDX_CHUNK), jnp.int32),
            pltpu.VMEM((BPW, D), jnp.float32),
            pltpu.VMEM((BPW, D), jnp.float32),
            pltpu.VMEM((2, 16), jnp.float32),
            pltpu.VMEM((BPW,), jnp.float32),
            pltpu.SemaphoreType.DMA,
            pltpu.SemaphoreType.DMA,
        ],
    )
    return fn(uidx, iidx, user_table, item_table, wb)


def kernel(user_idx, item_idx, user_table, item_table, W_aff, b_aff):
    uidx = user_idx.reshape(NW, NCHUNK, IDX_CHUNK).astype(jnp.int32)
    iidx = item_idx.reshape(NW, NCHUNK, IDX_CHUNK).astype(jnp.int32)
    wb = jnp.stack([jnp.full((16,), W_aff[0, 0], jnp.float32),
                    jnp.full((16,), b_aff[0], jnp.float32)])
    return _mf_call(uidx, iidx, user_table, item_table, wb)
